# baseline (speedup 1.0000x reference)
import functools

import jax
import jax.numpy as jnp
from jax import lax
from jax.experimental import pallas as pl
from jax.experimental.pallas import tpu as pltpu
from jax.experimental.pallas import tpu_sc as plsc

CODES = 1000000
WIDTH = 64
BATCH = 16384
FIELDS = 26

NC = 2
NS = 16
LANES = 16
NW = NC * NS
B = BATCH * FIELDS
BPW = B // NW
CHUNK = 128
NCHUNK = BPW // CHUNK
NBUF = 4
NGROUP = NCHUNK // NBUF

_mesh = plsc.VectorSubcoreMesh(
    core_axis_name="c", subcore_axis_name="s", num_cores=NC, num_subcores=NS
)


@functools.partial(
    pl.kernel,
    out_type=jax.ShapeDtypeStruct((B, WIDTH), jnp.float32),
    mesh=_mesh,
    compiler_params=pltpu.CompilerParams(use_tc_tiling_on_sc=False),
    scratch_types=[
        pltpu.VMEM((BPW,), jnp.float32),
        pltpu.VMEM((BPW,), jnp.int32),
        pltpu.VMEM((NBUF, CHUNK, WIDTH), jnp.float32),
        [pltpu.SemaphoreType.DMA] * NBUF,
    ],
)
def _embed_gather(x_hbm, tab_hbm, out_hbm, x_v, idx_v, rows_v, gsems):
    wid = lax.axis_index("s") * NC + lax.axis_index("c")
    base = wid * BPW

    pltpu.sync_copy(x_hbm.at[pl.ds(base, BPW)], x_v)

    def conv(i, carry):
        v = x_v[pl.ds(i * LANES, LANES)]
        v = jnp.where(v != v, jnp.float32(CODES), v)
        idx_v[pl.ds(i * LANES, LANES)] = v.astype(jnp.int32)
        return carry

    lax.fori_loop(0, BPW // LANES, conv, 0, unroll=4)

    def gather(j, b):
        return pltpu.make_async_copy(
            tab_hbm.at[idx_v.at[pl.ds(j * CHUNK, CHUNK)]], rows_v.at[b], gsems[b]
        )

    def write_out(j, b):
        pltpu.sync_copy(rows_v.at[b], out_hbm.at[pl.ds(base + j * CHUNK, CHUNK)])

    for b in range(NBUF):
        gather(b, b).start()

    def group(gi, carry):
        for b in range(NBUF):
            j = gi * NBUF + b
            gather(j, b).wait()
            write_out(j, b)
            gather(j + NBUF, b).start()
        return carry

    lax.fori_loop(0, NGROUP - 1, group, 0)

    for b in range(NBUF):
        j = (NGROUP - 1) * NBUF + b
        gather(j, b).wait()
        write_out(j, b)


def kernel(x, embed):
    out = _embed_gather(x.reshape(B), embed)
    return out.reshape(BATCH, FIELDS, WIDTH)
